# baseline (speedup 1.0000x reference)
import functools

import jax
import jax.numpy as jnp
from jax import lax
from jax.experimental import pallas as pl
from jax.experimental.pallas import tpu as pltpu
from jax.experimental.pallas import tpu_sc as plsc

LANE = 128
PAD = 128
CHUNK = 512


@functools.lru_cache(maxsize=None)
def _build(batch: int, hist: int, d: int):
  info = plsc.get_sparse_core_info()
  nw = info.num_cores * info.num_subcores
  assert batch % nw == 0
  b_per_w = batch // nw
  n_per_w = b_per_w * hist
  assert n_per_w % CHUNK == 0 and CHUNK % LANE == 0
  n_chunk = n_per_w // CHUNK
  assert n_chunk % 2 == 0
  n_half = n_chunk // 2
  n_sub = CHUNK // LANE
  h_starts = list(range(0, hist - 15, 16))
  if h_starts[-1] + 16 < hist:
    h_starts.append(hist - 16)

  mesh = plsc.VectorSubcoreMesh(core_axis_name="c", subcore_axis_name="s")

  @functools.partial(
      pl.kernel,
      out_type=jax.ShapeDtypeStruct((batch * hist, PAD), jnp.float32),
      mesh=mesh,
      scratch_types=[
          pltpu.VMEM((hist, b_per_w), jnp.int32),
          pltpu.VMEM((n_per_w,), jnp.int32),
          pltpu.VMEM((CHUNK, d), jnp.float32),
          pltpu.VMEM((CHUNK, d), jnp.float32),
          pltpu.SemaphoreType.DMA,
          pltpu.SemaphoreType.DMA,
          pltpu.SemaphoreType.DMA,
          pltpu.SemaphoreType.DMA,
      ],
      compiler_params=pltpu.CompilerParams(use_tc_tiling_on_sc=False,
                                           needs_layout_passes=False),
  )
  def gather_kernel(table_hbm, idx_hbm, out_hbm,
                    idx_slab, idx_dense, rows0, rows1, sg0, sg1, so0, so1):
    wid = lax.axis_index("s") * info.num_cores + lax.axis_index("c")
    col0 = wid * b_per_w
    flat_base = wid * n_per_w
    row_bufs = (rows0, rows1)
    sgs = (sg0, sg1)
    sos = (so0, so1)

    pltpu.sync_copy(idx_hbm.at[pl.ds(0, hist), pl.ds(col0, b_per_w)],
                    idx_slab)
    lane_iota = jax.lax.iota(jnp.int32, 16)

    def transpose_body(b, carry):
      b_ids = jnp.full((16,), b, jnp.int32)
      for h0 in h_starts:
        v = plsc.load_gather(idx_slab, [lane_iota + h0, b_ids])
        idx_dense[pl.ds(b * hist + h0, 16)] = v
      return carry

    lax.fori_loop(0, b_per_w, transpose_body, 0)

    def fire_gathers(c, buf):
      for j in range(n_sub):
        pltpu.async_copy(
            table_hbm.at[idx_dense.at[pl.ds(c * CHUNK + j * LANE, LANE)]],
            row_bufs[buf].at[pl.ds(j * LANE, LANE)],
            sgs[buf],
        )

    def drain_gathers(buf):
      pltpu.make_async_copy(out_hbm.at[pl.ds(0, CHUNK), pl.ds(0, d)],
                            row_bufs[buf], sgs[buf]).wait()

    def fire_wb(c, buf):
      pltpu.async_copy(
          row_bufs[buf],
          out_hbm.at[pl.ds(flat_base + c * CHUNK, CHUNK), pl.ds(0, d)],
          sos[buf])

    def drain_wb(buf):
      pltpu.make_async_copy(row_bufs[buf],
                            out_hbm.at[pl.ds(0, CHUNK), pl.ds(0, d)],
                            sos[buf]).wait()

    fire_gathers(0, 0)

    def body(i, carry):
      c0 = 2 * i

      @pl.when(i > 0)
      def _():
        drain_wb(1)

      fire_gathers(c0 + 1, 1)
      drain_gathers(0)
      fire_wb(c0, 0)

      drain_wb(0)

      @pl.when(i < n_half - 1)
      def _():
        fire_gathers(c0 + 2, 0)

      drain_gathers(1)
      fire_wb(c0 + 1, 1)
      return carry

    lax.fori_loop(0, n_half, body, 0)
    drain_wb(1)

  return gather_kernel


def kernel(item_feat_index, emb_table):
  batch, hist = item_feat_index.shape
  _, d = emb_table.shape
  idx = item_feat_index
  if idx.dtype != jnp.int32:
    idx = idx.astype(jnp.int32)
  padded = _build(batch, hist, d)(emb_table, idx.T)
  return padded.reshape(batch, hist, PAD)[:, :, :d]


# speedup vs baseline: 1.0459x; 1.0459x over previous
import functools

import jax
import jax.numpy as jnp
from jax import lax
from jax.experimental import pallas as pl
from jax.experimental.pallas import tpu as pltpu
from jax.experimental.pallas import tpu_sc as plsc

LANE = 128
PAD = 128
CHUNK = 512


@functools.lru_cache(maxsize=None)
def _build(n_flat: int, d: int):
  info = plsc.get_sparse_core_info()
  nw = info.num_cores * info.num_subcores
  assert n_flat % (nw * CHUNK) == 0 and CHUNK % LANE == 0
  n_per_w = n_flat // nw
  n_chunk = n_per_w // CHUNK
  assert n_chunk % 2 == 0
  n_half = n_chunk // 2
  n_sub = CHUNK // LANE

  mesh = plsc.VectorSubcoreMesh(core_axis_name="c", subcore_axis_name="s")

  @functools.partial(
      pl.kernel,
      out_type=jax.ShapeDtypeStruct((n_flat, PAD), jnp.float32),
      mesh=mesh,
      scratch_types=[
          pltpu.VMEM((CHUNK,), jnp.int32),
          pltpu.VMEM((CHUNK,), jnp.int32),
          pltpu.VMEM((CHUNK, d), jnp.float32),
          pltpu.VMEM((CHUNK, d), jnp.float32),
          pltpu.SemaphoreType.DMA,
          pltpu.SemaphoreType.DMA,
          pltpu.SemaphoreType.DMA,
          pltpu.SemaphoreType.DMA,
      ],
      compiler_params=pltpu.CompilerParams(use_tc_tiling_on_sc=False),
  )
  def gather_kernel(table_hbm, idx_hbm, out_hbm,
                    idx0, idx1, rows0, rows1, sg0, sg1, so0, so1):
    wid = lax.axis_index("s") * info.num_cores + lax.axis_index("c")
    flat_base = wid * n_per_w
    idx_bufs = (idx0, idx1)
    row_bufs = (rows0, rows1)
    sgs = (sg0, sg1)
    sos = (so0, so1)

    def fire_gathers(c, buf):
      pltpu.sync_copy(idx_hbm.at[pl.ds(flat_base + c * CHUNK, CHUNK)],
                      idx_bufs[buf])
      for j in range(n_sub):
        pltpu.async_copy(
            table_hbm.at[idx_bufs[buf].at[pl.ds(j * LANE, LANE)]],
            row_bufs[buf].at[pl.ds(j * LANE, LANE)],
            sgs[buf],
        )

    def drain_gathers(buf):
      pltpu.make_async_copy(out_hbm.at[pl.ds(0, CHUNK), pl.ds(0, d)],
                            row_bufs[buf], sgs[buf]).wait()

    def fire_wb(c, buf):
      pltpu.async_copy(
          row_bufs[buf],
          out_hbm.at[pl.ds(flat_base + c * CHUNK, CHUNK), pl.ds(0, d)],
          sos[buf])

    def drain_wb(buf):
      pltpu.make_async_copy(row_bufs[buf],
                            out_hbm.at[pl.ds(0, CHUNK), pl.ds(0, d)],
                            sos[buf]).wait()

    fire_gathers(0, 0)

    def body(i, carry):
      c0 = 2 * i

      @pl.when(i > 0)
      def _():
        drain_wb(1)

      fire_gathers(c0 + 1, 1)
      drain_gathers(0)
      fire_wb(c0, 0)

      drain_wb(0)

      @pl.when(i < n_half - 1)
      def _():
        fire_gathers(c0 + 2, 0)

      drain_gathers(1)
      fire_wb(c0 + 1, 1)
      return carry

    lax.fori_loop(0, n_half, body, 0)
    drain_wb(1)

  return gather_kernel


def kernel(item_feat_index, emb_table):
  batch, hist = item_feat_index.shape
  _, d = emb_table.shape
  idx = item_feat_index
  if idx.dtype != jnp.int32:
    idx = idx.astype(jnp.int32)
  idx_flat = idx.T.reshape(-1)
  padded = _build(batch * hist, d)(emb_table, idx_flat)
  return padded.reshape(hist, batch, PAD)[:, :, :d].transpose(1, 0, 2)


# speedup vs baseline: 1.0493x; 1.0033x over previous
import functools

import jax
import jax.numpy as jnp
from jax import lax
from jax.experimental import pallas as pl
from jax.experimental.pallas import tpu as pltpu
from jax.experimental.pallas import tpu_sc as plsc

LANE = 128
PAD = 128
CHUNK = 640


@functools.lru_cache(maxsize=None)
def _build(n_flat: int, d: int):
  info = plsc.get_sparse_core_info()
  nw = info.num_cores * info.num_subcores
  assert n_flat % (nw * CHUNK) == 0 and CHUNK % LANE == 0
  n_per_w = n_flat // nw
  n_chunk = n_per_w // CHUNK
  assert n_chunk % 2 == 0
  n_half = n_chunk // 2
  n_sub = CHUNK // LANE

  mesh = plsc.VectorSubcoreMesh(core_axis_name="c", subcore_axis_name="s")

  @functools.partial(
      pl.kernel,
      out_type=jax.ShapeDtypeStruct((n_flat, PAD), jnp.float32),
      mesh=mesh,
      scratch_types=[
          pltpu.VMEM((CHUNK,), jnp.int32),
          pltpu.VMEM((CHUNK,), jnp.int32),
          pltpu.VMEM((CHUNK, d), jnp.float32),
          pltpu.VMEM((CHUNK, d), jnp.float32),
          pltpu.SemaphoreType.DMA,
          pltpu.SemaphoreType.DMA,
          pltpu.SemaphoreType.DMA,
          pltpu.SemaphoreType.DMA,
      ],
      compiler_params=pltpu.CompilerParams(use_tc_tiling_on_sc=False),
  )
  def gather_kernel(table_hbm, idx_hbm, out_hbm,
                    idx0, idx1, rows0, rows1, sg0, sg1, so0, so1):
    wid = lax.axis_index("s") * info.num_cores + lax.axis_index("c")
    flat_base = wid * n_per_w
    idx_bufs = (idx0, idx1)
    row_bufs = (rows0, rows1)
    sgs = (sg0, sg1)
    sos = (so0, so1)

    def fire_gathers(c, buf):
      pltpu.sync_copy(idx_hbm.at[pl.ds(flat_base + c * CHUNK, CHUNK)],
                      idx_bufs[buf])
      for j in range(n_sub):
        pltpu.async_copy(
            table_hbm.at[idx_bufs[buf].at[pl.ds(j * LANE, LANE)]],
            row_bufs[buf].at[pl.ds(j * LANE, LANE)],
            sgs[buf],
        )

    def drain_gathers(buf):
      pltpu.make_async_copy(out_hbm.at[pl.ds(0, CHUNK), pl.ds(0, d)],
                            row_bufs[buf], sgs[buf]).wait()

    def fire_wb(c, buf):
      pltpu.async_copy(
          row_bufs[buf],
          out_hbm.at[pl.ds(flat_base + c * CHUNK, CHUNK), pl.ds(0, d)],
          sos[buf])

    def drain_wb(buf):
      pltpu.make_async_copy(row_bufs[buf],
                            out_hbm.at[pl.ds(0, CHUNK), pl.ds(0, d)],
                            sos[buf]).wait()

    fire_gathers(0, 0)

    def body(i, carry):
      c0 = 2 * i

      @pl.when(i > 0)
      def _():
        drain_wb(1)

      fire_gathers(c0 + 1, 1)
      drain_gathers(0)
      fire_wb(c0, 0)

      drain_wb(0)

      @pl.when(i < n_half - 1)
      def _():
        fire_gathers(c0 + 2, 0)

      drain_gathers(1)
      fire_wb(c0 + 1, 1)
      return carry

    lax.fori_loop(0, n_half, body, 0)
    drain_wb(1)

  return gather_kernel


def kernel(item_feat_index, emb_table):
  batch, hist = item_feat_index.shape
  _, d = emb_table.shape
  idx = item_feat_index
  if idx.dtype != jnp.int32:
    idx = idx.astype(jnp.int32)
  idx_flat = idx.T.reshape(-1)
  padded = _build(batch * hist, d)(emb_table, idx_flat)
  return padded.reshape(hist, batch, PAD)[:, :, :d].transpose(1, 0, 2)
